# Initial kernel scaffold; baseline (speedup 1.0000x reference)
import jax
import jax.numpy as jnp
from jax.experimental import pallas as pl


def kernel(x, adj, slab, drop_scale):
    raise NotImplementedError("write your pallas kernel here")


# speedup vs baseline: 1.3512x; 1.3512x over previous
import jax
import jax.numpy as jnp
from jax.experimental import pallas as pl
from jax.experimental.pallas import tpu as pltpu

_H = 32


def _dg(a, b, ca, cb):
    return jax.lax.dot_general(
        a, b, (((ca,), (cb,)), ((), ())), preferred_element_type=jnp.float32)


def _smg_body(x_ref, a_ref, d_ref, w0_ref, w1_ref, m10_ref, m11_ref,
              p1_ref, p2_ref, aux_ref, o_ref):
    G = x_ref.shape[0]
    aux = aux_ref[...]
    l12b0 = aux[0:64, 0:1]
    m1b0 = aux[0:32, 1:2]
    m2w0 = aux[0:32, 2:3]
    m2b0 = aux[0:1, 3:4]
    l12b1 = aux[0:64, 4:5]
    m1b1 = aux[0:32, 5:6]
    m2w1 = aux[0:32, 6:7]
    m2b1 = aux[0:1, 7:8]
    p1b = aux[0:32, 8:9]
    p2b = aux[0:32, 9:10]

    for g in range(G):
        xg = x_ref[g]
        A = a_ref[g]

        full0 = _dg(w0_ref[...], xg, 0, 1)
        hl0 = full0[0:64] + l12b0
        agg0 = _dg(hl0[0:32], A, 1, 1)
        cat0 = jnp.maximum(jnp.concatenate([agg0, hl0[32:64]], axis=0), 0.0)
        w0 = jnp.maximum(_dg(m10_ref[...], cat0, 0, 0) + m1b0, 0.0)
        s0 = jnp.sum(w0 * m2w0, axis=0, keepdims=True) + m2b0
        mask0 = jax.nn.sigmoid(s0)
        h0 = mask0 * full0[64:96]
        x1 = jnp.maximum((_dg(h0, A, 1, 1) + full0[96:128]) * mask0, 0.0)

        full1 = _dg(w1_ref[...], x1, 0, 0)
        hl1 = full1[0:64] * mask0 + l12b1
        agg1 = _dg(hl1[0:32], A, 1, 1)
        cat1 = jnp.maximum(jnp.concatenate([agg1, hl1[32:64]], axis=0), 0.0)
        w1 = jnp.maximum(_dg(m11_ref[...], cat1, 0, 0) + m1b1, 0.0)
        s1 = jnp.sum(w1 * m2w1, axis=0, keepdims=True) + m2b1
        mask1 = jax.nn.sigmoid(s1)
        h1 = mask1 * full1[64:96]
        x2 = jnp.maximum((_dg(h1, A, 1, 1) + full1[96:128]) * mask1, 0.0)

        y = jnp.maximum(_dg(p1_ref[...], x2, 0, 0) + p1b, 0.0)
        y = y * d_ref[g].T
        o_ref[g] = (_dg(p2_ref[...], y, 0, 0) + p2b).T


def kernel(x, adj, slab, drop_scale):
    b, n, ci = x.shape

    W0 = jnp.concatenate([slab[0:16, :64], slab[80:96, :64]], axis=1)
    W1 = jnp.concatenate([slab[96:128, :64], slab[192:224, :64]], axis=1)
    M10 = slab[16:80, :32]
    M11 = slab[128:192, :32]
    P1 = slab[224:256, :32]
    P2 = slab[256:288, :32]

    def col(off, m):
        return jnp.pad(slab[off, :m], (0, 64 - m))

    aux = jnp.stack([
        col(288, 64), col(296, 32), col(304, 32), col(312, 1),
        col(320, 64), col(328, 32), col(336, 32), col(344, 1),
        col(352, 32), col(360, 32),
    ], axis=1)
    aux = jnp.pad(aux, ((0, 0), (0, 6)))

    G = next(g for g in (8, 4, 2, 1) if b % g == 0)
    return pl.pallas_call(
        _smg_body,
        out_shape=jax.ShapeDtypeStruct((b, n, _H), jnp.float32),
        grid=(b // G,),
        in_specs=[
            pl.BlockSpec((G, n, ci), lambda i: (i, 0, 0)),
            pl.BlockSpec((G, n, n), lambda i: (i, 0, 0)),
            pl.BlockSpec((G, n, _H), lambda i: (i, 0, 0)),
            pl.BlockSpec((16, 128), lambda i: (0, 0)),
            pl.BlockSpec((32, 128), lambda i: (0, 0)),
            pl.BlockSpec((64, 32), lambda i: (0, 0)),
            pl.BlockSpec((64, 32), lambda i: (0, 0)),
            pl.BlockSpec((32, 32), lambda i: (0, 0)),
            pl.BlockSpec((32, 32), lambda i: (0, 0)),
            pl.BlockSpec((64, 16), lambda i: (0, 0)),
        ],
        out_specs=pl.BlockSpec((G, n, _H), lambda i: (i, 0, 0)),
        compiler_params=pltpu.CompilerParams(
            dimension_semantics=("parallel",)),
    )(x, adj, drop_scale, W0, W1, M10, M11, P1, P2, aux)


# speedup vs baseline: 3.3013x; 2.4432x over previous
import jax
import jax.numpy as jnp
from jax.experimental import pallas as pl
from jax.experimental.pallas import tpu as pltpu

_H = 32


def _dg(a, b, ca, cb):
    return jax.lax.dot_general(
        a, b, (((ca,), (cb,)), ((), ())), preferred_element_type=jnp.float32)


def _smg_body(x_ref, a_ref, d_ref, w0_ref, w1_ref, m10_ref, m11_ref,
              p1_ref, p2_ref, aux_ref, o_ref):
    G, n, ci = x_ref.shape
    aux = aux_ref[...]
    l12b0 = aux[0:64, 0:1]
    m1b0 = aux[0:32, 1:2]
    m2w0 = aux[0:32, 2:3]
    m2b0 = aux[0:1, 3:4]
    l12b1 = aux[0:64, 4:5]
    m1b1 = aux[0:32, 5:6]
    m2w1 = aux[0:32, 6:7]
    m2b1 = aux[0:1, 7:8]
    p1b = aux[0:32, 8:9]
    p2b = aux[0:32, 9:10]

    def adots(lhs):
        return jnp.concatenate(
            [_dg(lhs[:, g * n:(g + 1) * n], a_ref[g], 1, 1) for g in range(G)],
            axis=1)

    X = x_ref[...].reshape(G * n, ci)
    full0 = _dg(w0_ref[...], X, 0, 1)
    hl0 = full0[0:64] + l12b0
    agg0 = adots(hl0[0:32])
    cat0 = jnp.maximum(jnp.concatenate([agg0, hl0[32:64]], axis=0), 0.0)
    w0 = jnp.maximum(_dg(m10_ref[...], cat0, 0, 0) + m1b0, 0.0)
    s0 = jnp.sum(w0 * m2w0, axis=0, keepdims=True) + m2b0
    mask0 = jax.nn.sigmoid(s0)
    h0 = mask0 * full0[64:96]
    x1 = jnp.maximum((adots(h0) + full0[96:128]) * mask0, 0.0)

    full1 = _dg(w1_ref[...], x1, 0, 0)
    hl1 = full1[0:64] * mask0 + l12b1
    agg1 = adots(hl1[0:32])
    cat1 = jnp.maximum(jnp.concatenate([agg1, hl1[32:64]], axis=0), 0.0)
    w1 = jnp.maximum(_dg(m11_ref[...], cat1, 0, 0) + m1b1, 0.0)
    s1 = jnp.sum(w1 * m2w1, axis=0, keepdims=True) + m2b1
    mask1 = jax.nn.sigmoid(s1)
    h1 = mask1 * full1[64:96]
    x2 = jnp.maximum((adots(h1) + full1[96:128]) * mask1, 0.0)

    y = jnp.maximum(_dg(p1_ref[...], x2, 0, 0) + p1b, 0.0)
    dT = jnp.concatenate([d_ref[g].T for g in range(G)], axis=1)
    y = y * dT
    out = _dg(p2_ref[...], y, 0, 0) + p2b
    for g in range(G):
        o_ref[g] = out[:, g * n:(g + 1) * n].T


def kernel(x, adj, slab, drop_scale):
    b, n, ci = x.shape

    W0 = jnp.concatenate([slab[0:16, :64], slab[80:96, :64]], axis=1)
    W1 = jnp.concatenate([slab[96:128, :64], slab[192:224, :64]], axis=1)
    M10 = slab[16:80, :32]
    M11 = slab[128:192, :32]
    P1 = slab[224:256, :32]
    P2 = slab[256:288, :32]

    def col(off, m):
        return jnp.pad(slab[off, :m], (0, 64 - m))

    aux = jnp.stack([
        col(288, 64), col(296, 32), col(304, 32), col(312, 1),
        col(320, 64), col(328, 32), col(336, 32), col(344, 1),
        col(352, 32), col(360, 32),
    ], axis=1)
    aux = jnp.pad(aux, ((0, 0), (0, 6)))

    G = next(g for g in (8, 4, 2, 1) if b % g == 0)
    return pl.pallas_call(
        _smg_body,
        out_shape=jax.ShapeDtypeStruct((b, n, _H), jnp.float32),
        grid=(b // G,),
        in_specs=[
            pl.BlockSpec((G, n, ci), lambda i: (i, 0, 0)),
            pl.BlockSpec((G, n, n), lambda i: (i, 0, 0)),
            pl.BlockSpec((G, n, _H), lambda i: (i, 0, 0)),
            pl.BlockSpec((16, 128), lambda i: (0, 0)),
            pl.BlockSpec((32, 128), lambda i: (0, 0)),
            pl.BlockSpec((64, 32), lambda i: (0, 0)),
            pl.BlockSpec((64, 32), lambda i: (0, 0)),
            pl.BlockSpec((32, 32), lambda i: (0, 0)),
            pl.BlockSpec((32, 32), lambda i: (0, 0)),
            pl.BlockSpec((64, 16), lambda i: (0, 0)),
        ],
        out_specs=pl.BlockSpec((G, n, _H), lambda i: (i, 0, 0)),
        compiler_params=pltpu.CompilerParams(
            dimension_semantics=("parallel",)),
    )(x, adj, drop_scale, W0, W1, M10, M11, P1, P2, aux)


# speedup vs baseline: 3.5864x; 1.0864x over previous
import jax
import jax.numpy as jnp
from jax.experimental import pallas as pl
from jax.experimental.pallas import tpu as pltpu

_H = 32


def _dg(a, b, ca, cb):
    return jax.lax.dot_general(
        a, b, (((ca,), (cb,)), ((), ())), preferred_element_type=jnp.float32)


def _smg_body(x_ref, a_ref, d_ref, w0_ref, w1_ref, m10_ref, m11_ref,
              p1_ref, p2_ref, aux_ref, o_ref):
    G, n, ci = x_ref.shape
    aux = aux_ref[...]
    l12b0 = aux[0:64, 0:1]
    m1b0 = aux[0:32, 1:2]
    m2w0 = aux[0:32, 2:3]
    m2b0 = aux[0:1, 3:4]
    l12b1 = aux[0:64, 4:5]
    m1b1 = aux[0:32, 5:6]
    m2w1 = aux[0:32, 6:7]
    m2b1 = aux[0:1, 7:8]
    p1b = aux[0:32, 8:9]
    p2b = aux[0:32, 9:10]

    def adots(lhs):
        return jnp.concatenate(
            [_dg(lhs[:, g * n:(g + 1) * n], a_ref[g], 1, 1) for g in range(G)],
            axis=1)

    X = x_ref[...].reshape(G * n, ci)
    full0 = _dg(w0_ref[...], X, 0, 1)
    hl0 = full0[0:64] + l12b0
    agg0 = adots(hl0[0:32])
    cat0 = jnp.maximum(jnp.concatenate([agg0, hl0[32:64]], axis=0), 0.0)
    w0 = jnp.maximum(_dg(m10_ref[...], cat0, 0, 0) + m1b0, 0.0)
    s0 = jnp.sum(w0 * m2w0, axis=0, keepdims=True) + m2b0
    mask0 = jax.nn.sigmoid(s0)
    h0 = mask0 * full0[64:96]
    x1 = jnp.maximum((adots(h0) + full0[96:128]) * mask0, 0.0)

    full1 = _dg(w1_ref[...], x1, 0, 0)
    hl1 = full1[0:64] * mask0 + l12b1
    agg1 = adots(hl1[0:32])
    cat1 = jnp.maximum(jnp.concatenate([agg1, hl1[32:64]], axis=0), 0.0)
    w1 = jnp.maximum(_dg(m11_ref[...], cat1, 0, 0) + m1b1, 0.0)
    s1 = jnp.sum(w1 * m2w1, axis=0, keepdims=True) + m2b1
    mask1 = jax.nn.sigmoid(s1)
    h1 = mask1 * full1[64:96]
    x2 = jnp.maximum((adots(h1) + full1[96:128]) * mask1, 0.0)

    y = jnp.maximum(_dg(p1_ref[...], x2, 0, 0) + p1b, 0.0)
    dT = jnp.concatenate([d_ref[g].T for g in range(G)], axis=1)
    y = y * dT
    out = _dg(p2_ref[...], y, 0, 0) + p2b
    for g in range(G):
        o_ref[g] = out[:, g * n:(g + 1) * n].T


def kernel(x, adj, slab, drop_scale):
    b, n, ci = x.shape

    W0 = jnp.concatenate([slab[0:16, :64], slab[80:96, :64]], axis=1)
    W1 = jnp.concatenate([slab[96:128, :64], slab[192:224, :64]], axis=1)
    M10 = slab[16:80, :32]
    M11 = slab[128:192, :32]
    P1 = slab[224:256, :32]
    P2 = slab[256:288, :32]

    def col(off, m):
        return jnp.pad(slab[off, :m], (0, 64 - m))

    aux = jnp.stack([
        col(288, 64), col(296, 32), col(304, 32), col(312, 1),
        col(320, 64), col(328, 32), col(336, 32), col(344, 1),
        col(352, 32), col(360, 32),
    ], axis=1)
    aux = jnp.pad(aux, ((0, 0), (0, 6)))

    G = next(g for g in (16, 8, 4, 2, 1) if b % g == 0)
    return pl.pallas_call(
        _smg_body,
        out_shape=jax.ShapeDtypeStruct((b, n, _H), jnp.float32),
        grid=(b // G,),
        in_specs=[
            pl.BlockSpec((G, n, ci), lambda i: (i, 0, 0)),
            pl.BlockSpec((G, n, n), lambda i: (i, 0, 0)),
            pl.BlockSpec((G, n, _H), lambda i: (i, 0, 0)),
            pl.BlockSpec((16, 128), lambda i: (0, 0)),
            pl.BlockSpec((32, 128), lambda i: (0, 0)),
            pl.BlockSpec((64, 32), lambda i: (0, 0)),
            pl.BlockSpec((64, 32), lambda i: (0, 0)),
            pl.BlockSpec((32, 32), lambda i: (0, 0)),
            pl.BlockSpec((32, 32), lambda i: (0, 0)),
            pl.BlockSpec((64, 16), lambda i: (0, 0)),
        ],
        out_specs=pl.BlockSpec((G, n, _H), lambda i: (i, 0, 0)),
        compiler_params=pltpu.CompilerParams(
            dimension_semantics=("parallel",)),
    )(x, adj, drop_scale, W0, W1, M10, M11, P1, P2, aux)
